# Initial kernel scaffold; baseline (speedup 1.0000x reference)
import jax
import jax.numpy as jnp
from jax.experimental import pallas as pl


def kernel(feature, edge_index, edge_norm, W, b):
    raise NotImplementedError("write your pallas kernel here")


# speedup vs baseline: 2.3949x; 2.3949x over previous
import functools

import jax
import jax.numpy as jnp
from jax import lax
from jax.experimental import pallas as pl
from jax.experimental.pallas import tpu as pltpu
from jax.experimental.pallas import tpu_sc as plsc

N = 10000
N_PAD = 10240
E = 160000
D = 256
DH = 128

NC = 2
NS = 16
LANES = 16

ROWS_PER_TILE = N_PAD // NS
EDGES_PER_TILE_DEG = E // (NC * NS)
EDGES_PER_TILE_MSG = E // NS
G_DEG = 40
G_MSG = 80

def _mesh():
    return plsc.VectorSubcoreMesh(core_axis_name="c", subcore_axis_name="s",
                                  num_cores=NC, num_subcores=NS)


def _zeros16():
    return jnp.zeros((LANES,), jnp.float32)


def _splat(vec16, t):
    idx = lax.broadcast(t, (LANES,))
    dnums = lax.GatherDimensionNumbers(
        offset_dims=(), collapsed_slice_dims=(0,), start_index_map=(0,))
    return lax.gather(vec16, idx.reshape(LANES, 1), dimension_numbers=dnums,
                      slice_sizes=(1,),
                      mode=lax.GatherScatterMode.PROMISE_IN_BOUNDS)


@functools.lru_cache(maxsize=None)
def _sc_deg_call():
    return pl.kernel(
        _sc_deg_body,
        out_type=jax.ShapeDtypeStruct((NC * N_PAD, DH), jnp.float32),
        mesh=_mesh(),
        scratch_types=[
            pltpu.VMEM_SHARED((N_PAD, DH), jnp.float32),
            pltpu.VMEM((2 * G_DEG, DH), jnp.float32),
            pltpu.VMEM((G_DEG, DH), jnp.float32),
            pltpu.VMEM((G_DEG,), jnp.int32),
        ],
    )


def _sc_deg_body(dst_hbm, deg_hbm, shared, buf_v, ones_v, idx_v):
    c = lax.axis_index("c")
    s = lax.axis_index("s")
    w = s * NC + c

    @pl.loop(0, 2 * G_DEG)
    def _(i):
        for k in range(DH // LANES):
            buf_v[i, pl.ds(k * LANES, LANES)] = _zeros16()

    @pl.loop(0, G_DEG)
    def _(i):
        for k in range(DH // LANES):
            ones_v[i, pl.ds(k * LANES, LANES)] = _zeros16()

    n_copies = ROWS_PER_TILE // (2 * G_DEG)

    @pl.loop(0, n_copies)
    def _(i):
        pltpu.sync_copy(
            buf_v, shared.at[pl.ds(s * ROWS_PER_TILE + i * 2 * G_DEG, 2 * G_DEG)])

    plsc.subcore_barrier()

    iota = lax.broadcasted_iota(jnp.int32, (LANES,), 0)
    e0 = jnp.where(iota == 0, 1.0, 0.0).astype(jnp.float32)

    @pl.loop(0, G_DEG)
    def _(i):
        ones_v[i, pl.ds(0, LANES)] = e0

    @pl.loop(0, EDGES_PER_TILE_DEG // G_DEG)
    def _(i):
        base = w * EDGES_PER_TILE_DEG + i * G_DEG
        pltpu.sync_copy(dst_hbm.at[pl.ds(base, G_DEG)], idx_v)
        pltpu.sync_copy(ones_v, shared.at[idx_v], add=True)

    plsc.subcore_barrier()

    @pl.loop(0, n_copies)
    def _(i):
        off = s * ROWS_PER_TILE + i * 2 * G_DEG
        pltpu.sync_copy(shared.at[pl.ds(off, 2 * G_DEG)],
                        deg_hbm.at[pl.ds(c * N_PAD + off, 2 * G_DEG)])


@functools.lru_cache(maxsize=None)
def _sc_msg_call():
    return pl.kernel(
        _sc_msg_body,
        out_type=(jax.ShapeDtypeStruct((N_PAD, DH), jnp.float32),
                  jax.ShapeDtypeStruct((N_PAD, DH), jnp.float32)),
        mesh=_mesh(),
        scratch_types=[
            pltpu.VMEM_SHARED((N_PAD, DH), jnp.float32),
            pltpu.VMEM((G_MSG, DH), jnp.float32),
            pltpu.VMEM((G_MSG,), jnp.int32),
            pltpu.VMEM((G_MSG,), jnp.int32),
            pltpu.VMEM((G_MSG,), jnp.float32),
        ],
    )


def _sc_msg_body(h0_hbm, h1_hbm, src_hbm, dst_hbm, en_hbm, agg0_hbm, agg1_hbm,
                 shared, rows_v, src_v, dst_v, norm_v):
    c = lax.axis_index("c")
    s = lax.axis_index("s")

    @pl.loop(0, G_MSG)
    def _(i):
        for k in range(DH // LANES):
            rows_v[i, pl.ds(k * LANES, LANES)] = _zeros16()

    n_copies = ROWS_PER_TILE // G_MSG

    @pl.loop(0, n_copies)
    def _(i):
        pltpu.sync_copy(
            rows_v, shared.at[pl.ds(s * ROWS_PER_TILE + i * G_MSG, G_MSG)])

    plsc.subcore_barrier()

    def run(h_hbm, agg_hbm):
        @pl.loop(0, EDGES_PER_TILE_MSG // G_MSG)
        def _(i):
            base = s * EDGES_PER_TILE_MSG + i * G_MSG
            pltpu.sync_copy(src_hbm.at[pl.ds(base, G_MSG)], src_v)
            pltpu.sync_copy(dst_hbm.at[pl.ds(base, G_MSG)], dst_v)
            pltpu.sync_copy(en_hbm.at[pl.ds(base, G_MSG)], norm_v)
            pltpu.sync_copy(h_hbm.at[src_v], rows_v)

            @pl.loop(0, G_MSG)
            def _(g):
                jbase = pl.multiple_of((g // LANES) * LANES, LANES)
                n16 = norm_v[pl.ds(jbase, LANES)]
                spl = _splat(n16, g - jbase)
                for k in range(DH // LANES):
                    sl = pl.ds(k * LANES, LANES)
                    rows_v[g, sl] = rows_v[g, sl] * spl

            pltpu.sync_copy(rows_v, shared.at[dst_v], add=True)

        plsc.subcore_barrier()

        @pl.loop(0, n_copies)
        def _(i):
            off = s * ROWS_PER_TILE + i * G_MSG
            pltpu.sync_copy(shared.at[pl.ds(off, G_MSG)],
                            agg_hbm.at[pl.ds(off, G_MSG)])

    @pl.when(c == 0)
    def _():
        run(h0_hbm, agg0_hbm)

    @pl.when(c == 1)
    def _():
        run(h1_hbm, agg1_hbm)


_TC_R = 1024


def _norm_block(degp):
    d = degp[0, :, 0] + degp[1, :, 0]
    return lax.rsqrt(jnp.maximum(d, 1.0))


def _tc_linear_body(x_ref, wt_ref, b_ref, degp_ref, h0_ref, h1_ref):
    h = jnp.dot(x_ref[...], wt_ref[...], preferred_element_type=jnp.float32)
    h = h + b_ref[...]
    h = h * _norm_block(degp_ref[...])[:, None]
    h0_ref[...] = h[:, :DH]
    h1_ref[...] = h[:, DH:]


def _tc_linear(feat_p, wt, b2, degp):
    grid = (N_PAD // _TC_R,)
    return pl.pallas_call(
        _tc_linear_body,
        grid=grid,
        in_specs=[
            pl.BlockSpec((_TC_R, D), lambda i: (i, 0)),
            pl.BlockSpec((D, D), lambda i: (0, 0)),
            pl.BlockSpec((1, D), lambda i: (0, 0)),
            pl.BlockSpec((NC, _TC_R, DH), lambda i: (0, i, 0)),
        ],
        out_specs=(pl.BlockSpec((_TC_R, DH), lambda i: (i, 0)),
                   pl.BlockSpec((_TC_R, DH), lambda i: (i, 0))),
        out_shape=(jax.ShapeDtypeStruct((N_PAD, DH), jnp.float32),
                   jax.ShapeDtypeStruct((N_PAD, DH), jnp.float32)),
    )(feat_p, wt, b2, degp)


def _tc_post_body(a0_ref, a1_ref, degp_ref, out_ref):
    norm = _norm_block(degp_ref[...])[:, None]
    out_ref[...] = jnp.concatenate([a0_ref[...] * norm, a1_ref[...] * norm],
                                   axis=1)


def _tc_post(agg0, agg1, degp):
    grid = (N_PAD // _TC_R,)
    return pl.pallas_call(
        _tc_post_body,
        grid=grid,
        in_specs=[
            pl.BlockSpec((_TC_R, DH), lambda i: (i, 0)),
            pl.BlockSpec((_TC_R, DH), lambda i: (i, 0)),
            pl.BlockSpec((NC, _TC_R, DH), lambda i: (0, i, 0)),
        ],
        out_specs=pl.BlockSpec((_TC_R, D), lambda i: (i, 0)),
        out_shape=jax.ShapeDtypeStruct((N_PAD, D), jnp.float32),
    )(agg0, agg1, degp)


@jax.jit
def kernel(feature, edge_index, edge_norm, W, b):
    src = edge_index[0].astype(jnp.int32)
    dst = edge_index[1].astype(jnp.int32)
    feat_p = jnp.pad(feature, ((0, N_PAD - N), (0, 0)))
    degp = _sc_deg_call()(dst).reshape(NC, N_PAD, DH)
    h0, h1 = _tc_linear(feat_p, W.T, b.reshape(1, D), degp)
    agg0, agg1 = _sc_msg_call()(h0, h1, src, dst, edge_norm)
    out = _tc_post(agg0, agg1, degp)
    return out[:N]


# speedup vs baseline: 3.4853x; 1.4553x over previous
import functools

import jax
import jax.numpy as jnp
from jax import lax
from jax.experimental import pallas as pl
from jax.experimental.pallas import tpu as pltpu
from jax.experimental.pallas import tpu_sc as plsc

N = 10000
N_PAD = 10240
E = 160000
D = 256
DH = 128

NC = 2
NS = 16
LANES = 16

ROWS_PER_TILE = N_PAD // NS
DW = 16

DEG_PER_TILE = 5120
G_DEG = 128

G_MSG = 96
MSG_CHUNKS = 106
MSG_SLAB_CHUNKS = MSG_CHUNKS + 1
MSG_PER_TILE = MSG_SLAB_CHUNKS * G_MSG
E_MSG_PAD = NS * MSG_PER_TILE


def _mesh():
    return plsc.VectorSubcoreMesh(core_axis_name="c", subcore_axis_name="s",
                                  num_cores=NC, num_subcores=NS)


def _zeros16():
    return jnp.zeros((LANES,), jnp.float32)


def _splat(vec16, t):
    idx = lax.broadcast(t, (LANES,))
    dnums = lax.GatherDimensionNumbers(
        offset_dims=(), collapsed_slice_dims=(0,), start_index_map=(0,))
    return lax.gather(vec16, idx.reshape(LANES, 1), dimension_numbers=dnums,
                      slice_sizes=(1,),
                      mode=lax.GatherScatterMode.PROMISE_IN_BOUNDS)


@functools.lru_cache(maxsize=None)
def _sc_deg_call():
    return pl.kernel(
        _sc_deg_body,
        out_type=jax.ShapeDtypeStruct((NC * N_PAD, DW), jnp.float32),
        mesh=_mesh(),
        scratch_types=[
            pltpu.VMEM_SHARED((N_PAD, DW), jnp.float32),
            pltpu.VMEM((ROWS_PER_TILE, DW), jnp.float32),
            pltpu.VMEM((G_DEG, DW), jnp.float32),
            pltpu.VMEM((DEG_PER_TILE,), jnp.int32),
            pltpu.VMEM((G_DEG,), jnp.int32),
        ],
    )


def _sc_deg_body(dst_hbm, deg_hbm, shared, zero_v, ones_v, slab_v, idx_v):
    c = lax.axis_index("c")
    s = lax.axis_index("s")
    w = s * NC + c

    pltpu.sync_copy(dst_hbm.at[pl.ds(w * DEG_PER_TILE, DEG_PER_TILE)], slab_v)

    iota = lax.broadcasted_iota(jnp.int32, (LANES,), 0)
    e0 = jnp.where(iota == 0, 1.0, 0.0).astype(jnp.float32)

    @pl.loop(0, G_DEG)
    def _(i):
        ones_v[i, :] = e0

    @pl.loop(0, ROWS_PER_TILE)
    def _(i):
        zero_v[i, :] = _zeros16()

    pltpu.sync_copy(zero_v, shared.at[pl.ds(s * ROWS_PER_TILE, ROWS_PER_TILE)])
    plsc.subcore_barrier()

    @pl.loop(0, DEG_PER_TILE // G_DEG)
    def _(i):
        for j in range(G_DEG // LANES):
            sl = pl.ds(j * LANES, LANES)
            idx_v[sl] = slab_v[pl.ds(i * G_DEG + j * LANES, LANES)]
        pltpu.sync_copy(ones_v, shared.at[idx_v], add=True)

    plsc.subcore_barrier()

    off = s * ROWS_PER_TILE
    pltpu.sync_copy(shared.at[pl.ds(off, ROWS_PER_TILE)],
                    deg_hbm.at[pl.ds(c * N_PAD + off, ROWS_PER_TILE)])


@functools.lru_cache(maxsize=None)
def _sc_msg_call():
    return pl.kernel(
        _sc_msg_body,
        out_type=(jax.ShapeDtypeStruct((N_PAD, DH), jnp.float32),
                  jax.ShapeDtypeStruct((N_PAD, DH), jnp.float32)),
        mesh=_mesh(),
        scratch_types=[
            pltpu.VMEM_SHARED((N_PAD, DH), jnp.float32),
            pltpu.VMEM((G_MSG, DH), jnp.float32),
            pltpu.VMEM((MSG_PER_TILE,), jnp.int32),
            pltpu.VMEM((MSG_PER_TILE,), jnp.int32),
            pltpu.VMEM((MSG_PER_TILE,), jnp.float32),
            pltpu.VMEM((G_MSG,), jnp.int32),
            pltpu.VMEM((G_MSG,), jnp.int32),
        ],
    )


def _sc_msg_body(h0_hbm, h1_hbm, src_hbm, dst_hbm, en_hbm, agg0_hbm, agg1_hbm,
                 shared, rows_v, srcs_v, dsts_v, ens_v, src_v, dst_v):
    c = lax.axis_index("c")
    s = lax.axis_index("s")

    base = s * MSG_PER_TILE
    pltpu.sync_copy(src_hbm.at[pl.ds(base, MSG_PER_TILE)], srcs_v)
    pltpu.sync_copy(dst_hbm.at[pl.ds(base, MSG_PER_TILE)], dsts_v)
    pltpu.sync_copy(en_hbm.at[pl.ds(base, MSG_PER_TILE)], ens_v)

    @pl.loop(0, G_MSG)
    def _(i):
        for k in range(DH // LANES):
            rows_v[i, pl.ds(k * LANES, LANES)] = _zeros16()

    @pl.loop(0, ROWS_PER_TILE // G_MSG)
    def _(i):
        pltpu.sync_copy(
            rows_v, shared.at[pl.ds(s * ROWS_PER_TILE + i * G_MSG, G_MSG)])

    rem = ROWS_PER_TILE - (ROWS_PER_TILE // G_MSG) * G_MSG
    pltpu.sync_copy(
        rows_v.at[pl.ds(0, rem)],
        shared.at[pl.ds(s * ROWS_PER_TILE + ROWS_PER_TILE - rem, rem)])

    plsc.subcore_barrier()

    def run(h_hbm, agg_hbm):
        @pl.loop(0, MSG_CHUNKS)
        def _(ci):
            cbase = ci * G_MSG
            for j in range(G_MSG // LANES):
                sl = pl.ds(j * LANES, LANES)
                src_v[sl] = srcs_v[pl.ds(cbase + j * LANES, LANES)]
                dst_v[sl] = dsts_v[pl.ds(cbase + j * LANES, LANES)]
            pltpu.sync_copy(h_hbm.at[src_v], rows_v)

            @pl.loop(0, G_MSG // LANES)
            def _(j):
                w16 = ens_v[pl.ds(cbase + j * LANES, LANES)]
                for t in range(LANES):
                    spl = _splat(w16, t)
                    r = j * LANES + t
                    for k in range(DH // LANES):
                        sl = pl.ds(k * LANES, LANES)
                        rows_v[r, sl] = rows_v[r, sl] * spl

            pltpu.sync_copy(rows_v, shared.at[dst_v], add=True)

        plsc.subcore_barrier()

        @pl.loop(0, ROWS_PER_TILE // 128)
        def _(i):
            off = s * ROWS_PER_TILE + i * 128
            pltpu.sync_copy(shared.at[pl.ds(off, 128)],
                            agg_hbm.at[pl.ds(off, 128)])

    @pl.when(c == 0)
    def _():
        run(h0_hbm, agg0_hbm)

    @pl.when(c == 1)
    def _():
        run(h1_hbm, agg1_hbm)


_TC_R = 1024


def _norm_block(degp):
    d = degp[0, :, 0] + degp[1, :, 0]
    return lax.rsqrt(jnp.maximum(d, 1.0))


def _tc_linear_body(x_ref, wt_ref, b_ref, degp_ref, h0_ref, h1_ref):
    h = jnp.dot(x_ref[...], wt_ref[...], preferred_element_type=jnp.float32)
    h = h + b_ref[...]
    h = h * _norm_block(degp_ref[...])[:, None]
    h0_ref[...] = h[:, :DH]
    h1_ref[...] = h[:, DH:]


def _tc_linear(feat_p, wt, b2, degp):
    grid = (N_PAD // _TC_R,)
    return pl.pallas_call(
        _tc_linear_body,
        grid=grid,
        in_specs=[
            pl.BlockSpec((_TC_R, D), lambda i: (i, 0)),
            pl.BlockSpec((D, D), lambda i: (0, 0)),
            pl.BlockSpec((1, D), lambda i: (0, 0)),
            pl.BlockSpec((NC, _TC_R, DW), lambda i: (0, i, 0)),
        ],
        out_specs=(pl.BlockSpec((_TC_R, DH), lambda i: (i, 0)),
                   pl.BlockSpec((_TC_R, DH), lambda i: (i, 0))),
        out_shape=(jax.ShapeDtypeStruct((N_PAD, DH), jnp.float32),
                   jax.ShapeDtypeStruct((N_PAD, DH), jnp.float32)),
    )(feat_p, wt, b2, degp)


def _tc_post_body(a0_ref, a1_ref, degp_ref, out_ref):
    norm = _norm_block(degp_ref[...])[:, None]
    out_ref[...] = jnp.concatenate([a0_ref[...] * norm, a1_ref[...] * norm],
                                   axis=1)


def _tc_post(agg0, agg1, degp):
    grid = (N_PAD // _TC_R,)
    return pl.pallas_call(
        _tc_post_body,
        grid=grid,
        in_specs=[
            pl.BlockSpec((_TC_R, DH), lambda i: (i, 0)),
            pl.BlockSpec((_TC_R, DH), lambda i: (i, 0)),
            pl.BlockSpec((NC, _TC_R, DW), lambda i: (0, i, 0)),
        ],
        out_specs=pl.BlockSpec((_TC_R, D), lambda i: (i, 0)),
        out_shape=jax.ShapeDtypeStruct((N_PAD, D), jnp.float32),
    )(agg0, agg1, degp)


def _pad_slab(arr, per_real, per_pad, fill):
    nt = arr.shape[0] // per_real
    a = arr.reshape(nt, per_real)
    return jnp.pad(a, ((0, 0), (0, per_pad - per_real)),
                   constant_values=fill).reshape(-1)


@jax.jit
def kernel(feature, edge_index, edge_norm, W, b):
    src = edge_index[0].astype(jnp.int32)
    dst = edge_index[1].astype(jnp.int32)
    feat_p = jnp.pad(feature, ((0, N_PAD - N), (0, 0)))

    dst_deg = _pad_slab(dst, E // (NC * NS), DEG_PER_TILE, N_PAD - 1)
    src_m = _pad_slab(src, E // NS, MSG_PER_TILE, 0)
    dst_m = _pad_slab(dst, E // NS, MSG_PER_TILE, N_PAD - 1)
    en_m = _pad_slab(edge_norm, E // NS, MSG_PER_TILE, 0.0)

    degp = _sc_deg_call()(dst_deg).reshape(NC, N_PAD, DW)
    h0, h1 = _tc_linear(feat_p, W.T, b.reshape(1, D), degp)
    agg0, agg1 = _sc_msg_call()(h0, h1, src_m, dst_m, en_m)
    out = _tc_post(agg0, agg1, degp)
    return out[:N]


# speedup vs baseline: 3.9187x; 1.1243x over previous
import functools

import jax
import jax.numpy as jnp
from jax import lax
from jax.experimental import pallas as pl
from jax.experimental.pallas import tpu as pltpu
from jax.experimental.pallas import tpu_sc as plsc

N = 10000
N_PAD = 10240
E = 160000
D = 256
DH = 128

NC = 2
NS = 16
LANES = 16

ROWS_PER_TILE = N_PAD // NS
DW = 16

DEG_PER_TILE = 5120
G_DEG = 128

G_MSG = 96
MSG_CHUNKS = 106
MSG_SLAB_CHUNKS = MSG_CHUNKS + 1
MSG_PER_TILE = MSG_SLAB_CHUNKS * G_MSG
E_MSG_PAD = NS * MSG_PER_TILE


def _mesh():
    return plsc.VectorSubcoreMesh(core_axis_name="c", subcore_axis_name="s",
                                  num_cores=NC, num_subcores=NS)


def _zeros16():
    return jnp.zeros((LANES,), jnp.float32)


def _splat(vec16, t):
    idx = lax.broadcast(t, (LANES,))
    dnums = lax.GatherDimensionNumbers(
        offset_dims=(), collapsed_slice_dims=(0,), start_index_map=(0,))
    return lax.gather(vec16, idx.reshape(LANES, 1), dimension_numbers=dnums,
                      slice_sizes=(1,),
                      mode=lax.GatherScatterMode.PROMISE_IN_BOUNDS)


@functools.lru_cache(maxsize=None)
def _sc_deg_call():
    return pl.kernel(
        _sc_deg_body,
        out_type=jax.ShapeDtypeStruct((NC * N_PAD, DW), jnp.float32),
        mesh=_mesh(),
        scratch_types=[
            pltpu.VMEM_SHARED((N_PAD, DW), jnp.float32),
            pltpu.VMEM((ROWS_PER_TILE, DW), jnp.float32),
            pltpu.VMEM((G_DEG, DW), jnp.float32),
            pltpu.VMEM((DEG_PER_TILE,), jnp.int32),
            pltpu.VMEM((G_DEG,), jnp.int32),
        ],
    )


def _sc_deg_body(dst_hbm, deg_hbm, shared, zero_v, ones_v, slab_v, idx_v):
    c = lax.axis_index("c")
    s = lax.axis_index("s")
    w = s * NC + c

    pltpu.sync_copy(dst_hbm.at[pl.ds(w * DEG_PER_TILE, DEG_PER_TILE)], slab_v)

    iota = lax.broadcasted_iota(jnp.int32, (LANES,), 0)
    e0 = jnp.where(iota == 0, 1.0, 0.0).astype(jnp.float32)

    @pl.loop(0, G_DEG)
    def _(i):
        ones_v[i, :] = e0

    @pl.loop(0, ROWS_PER_TILE)
    def _(i):
        zero_v[i, :] = _zeros16()

    pltpu.sync_copy(zero_v, shared.at[pl.ds(s * ROWS_PER_TILE, ROWS_PER_TILE)])
    plsc.subcore_barrier()

    @pl.loop(0, DEG_PER_TILE // G_DEG)
    def _(i):
        for j in range(G_DEG // LANES):
            sl = pl.ds(j * LANES, LANES)
            idx_v[sl] = slab_v[pl.ds(i * G_DEG + j * LANES, LANES)]
        pltpu.sync_copy(ones_v, shared.at[idx_v], add=True)

    plsc.subcore_barrier()

    off = s * ROWS_PER_TILE
    pltpu.sync_copy(shared.at[pl.ds(off, ROWS_PER_TILE)],
                    deg_hbm.at[pl.ds(c * N_PAD + off, ROWS_PER_TILE)])


@functools.lru_cache(maxsize=None)
def _sc_msg_call():
    return pl.kernel(
        _sc_msg_body,
        out_type=(jax.ShapeDtypeStruct((N_PAD, DH), jnp.float32),
                  jax.ShapeDtypeStruct((N_PAD, DH), jnp.float32)),
        mesh=_mesh(),
        scratch_types=[
            pltpu.VMEM_SHARED((N_PAD, DH), jnp.float32),
            pltpu.VMEM((G_MSG, DH), jnp.float32),
            pltpu.VMEM((G_MSG, DH), jnp.float32),
            pltpu.VMEM((MSG_PER_TILE,), jnp.int32),
            pltpu.VMEM((MSG_PER_TILE,), jnp.float32),
            pltpu.VMEM((G_MSG,), jnp.int32),
            pltpu.VMEM((G_MSG,), jnp.int32),
            pltpu.VMEM((G_MSG,), jnp.int32),
            pltpu.VMEM((G_MSG,), jnp.int32),
            pltpu.SemaphoreType.DMA,
            pltpu.SemaphoreType.DMA,
        ],
    )


def _sc_msg_body(h0_hbm, h1_hbm, src_hbm, dst_hbm, en_hbm, agg0_hbm, agg1_hbm,
                 shared, rows0_v, rows1_v, srcs_v, ens_v, src0_v, src1_v,
                 dst0_v, dst1_v, semg0, semg1):
    c = lax.axis_index("c")
    s = lax.axis_index("s")

    base = s * MSG_PER_TILE
    pltpu.sync_copy(src_hbm.at[pl.ds(base, MSG_PER_TILE)], srcs_v)
    pltpu.sync_copy(en_hbm.at[pl.ds(base, MSG_PER_TILE)], ens_v)

    @pl.loop(0, G_MSG)
    def _(i):
        for k in range(DH // LANES):
            rows0_v[i, pl.ds(k * LANES, LANES)] = _zeros16()

    @pl.loop(0, ROWS_PER_TILE // G_MSG)
    def _(i):
        pltpu.sync_copy(
            rows0_v, shared.at[pl.ds(s * ROWS_PER_TILE + i * G_MSG, G_MSG)])

    rem = ROWS_PER_TILE - (ROWS_PER_TILE // G_MSG) * G_MSG
    pltpu.sync_copy(
        rows0_v.at[pl.ds(0, rem)],
        shared.at[pl.ds(s * ROWS_PER_TILE + ROWS_PER_TILE - rem, rem)])

    plsc.subcore_barrier()

    def run(h_hbm, agg_hbm):
        bufs = ((rows0_v, src0_v, dst0_v, semg0),
                (rows1_v, src1_v, dst1_v, semg1))

        def copy_src_chunk(ci, src_v):
            cb = ci * G_MSG
            for j in range(G_MSG // LANES):
                sl = pl.ds(j * LANES, LANES)
                src_v[sl] = srcs_v[pl.ds(cb + j * LANES, LANES)]

        def prefetch(ci, rows_v, src_v, dst_v, semg):
            copy_src_chunk(ci, src_v)
            pltpu.async_copy(h_hbm.at[src_v], rows_v, semg)
            pltpu.async_copy(
                dst_hbm.at[pl.ds(base + ci * G_MSG, G_MSG)], dst_v, semg)

        def wait_prefetch(rows_v, src_v, dst_v, semg):
            pltpu.make_async_copy(h_hbm.at[src_v], rows_v, semg).wait()
            pltpu.make_async_copy(dst_hbm.at[pl.ds(0, G_MSG)], dst_v,
                                  semg).wait()

        def scale(ci, rows_v):
            @pl.loop(0, G_MSG // LANES)
            def _(j):
                w16 = ens_v[pl.ds(ci * G_MSG + j * LANES, LANES)]
                for t in range(LANES):
                    spl = _splat(w16, t)
                    r = j * LANES + t
                    for k in range(DH // LANES):
                        sl = pl.ds(k * LANES, LANES)
                        rows_v[r, sl] = rows_v[r, sl] * spl

        prefetch(0, *bufs[0])

        @pl.loop(0, MSG_CHUNKS // 2)
        def _(i):
            for b in (0, 1):
                rows_v, src_v, dst_v, semg = bufs[b]
                rows_o, src_o, dst_o, semg_o = bufs[1 - b]
                ci = i * 2 + b
                prefetch(ci + 1, rows_o, src_o, dst_o, semg_o)
                wait_prefetch(rows_v, src_v, dst_v, semg)
                scale(ci, rows_v)
                pltpu.sync_copy(rows_v, shared.at[dst_v], add=True)

        wait_prefetch(*bufs[0])

        plsc.subcore_barrier()

        @pl.loop(0, ROWS_PER_TILE // 128)
        def _(i):
            off = s * ROWS_PER_TILE + i * 128
            pltpu.sync_copy(shared.at[pl.ds(off, 128)],
                            agg_hbm.at[pl.ds(off, 128)])

    @pl.when(c == 0)
    def _():
        run(h0_hbm, agg0_hbm)

    @pl.when(c == 1)
    def _():
        run(h1_hbm, agg1_hbm)


_TC_R = 1024


def _norm_block(degp):
    d = degp[0, :, 0] + degp[1, :, 0]
    return lax.rsqrt(jnp.maximum(d, 1.0))


def _tc_linear_body(x_ref, wt_ref, b_ref, degp_ref, h0_ref, h1_ref):
    h = jnp.dot(x_ref[...], wt_ref[...], preferred_element_type=jnp.float32)
    h = h + b_ref[...]
    h = h * _norm_block(degp_ref[...])[:, None]
    h0_ref[...] = h[:, :DH]
    h1_ref[...] = h[:, DH:]


def _tc_linear(feat_p, wt, b2, degp):
    grid = (N_PAD // _TC_R,)
    return pl.pallas_call(
        _tc_linear_body,
        grid=grid,
        in_specs=[
            pl.BlockSpec((_TC_R, D), lambda i: (i, 0)),
            pl.BlockSpec((D, D), lambda i: (0, 0)),
            pl.BlockSpec((1, D), lambda i: (0, 0)),
            pl.BlockSpec((NC, _TC_R, DW), lambda i: (0, i, 0)),
        ],
        out_specs=(pl.BlockSpec((_TC_R, DH), lambda i: (i, 0)),
                   pl.BlockSpec((_TC_R, DH), lambda i: (i, 0))),
        out_shape=(jax.ShapeDtypeStruct((N_PAD, DH), jnp.float32),
                   jax.ShapeDtypeStruct((N_PAD, DH), jnp.float32)),
    )(feat_p, wt, b2, degp)


def _tc_post_body(a0_ref, a1_ref, degp_ref, out_ref):
    norm = _norm_block(degp_ref[...])[:, None]
    out_ref[...] = jnp.concatenate([a0_ref[...] * norm, a1_ref[...] * norm],
                                   axis=1)


def _tc_post(agg0, agg1, degp):
    grid = (N_PAD // _TC_R,)
    return pl.pallas_call(
        _tc_post_body,
        grid=grid,
        in_specs=[
            pl.BlockSpec((_TC_R, DH), lambda i: (i, 0)),
            pl.BlockSpec((_TC_R, DH), lambda i: (i, 0)),
            pl.BlockSpec((NC, _TC_R, DW), lambda i: (0, i, 0)),
        ],
        out_specs=pl.BlockSpec((_TC_R, D), lambda i: (i, 0)),
        out_shape=jax.ShapeDtypeStruct((N_PAD, D), jnp.float32),
    )(agg0, agg1, degp)


def _pad_slab(arr, per_real, per_pad, fill):
    nt = arr.shape[0] // per_real
    a = arr.reshape(nt, per_real)
    return jnp.pad(a, ((0, 0), (0, per_pad - per_real)),
                   constant_values=fill).reshape(-1)


@jax.jit
def kernel(feature, edge_index, edge_norm, W, b):
    src = edge_index[0].astype(jnp.int32)
    dst = edge_index[1].astype(jnp.int32)
    feat_p = jnp.pad(feature, ((0, N_PAD - N), (0, 0)))

    dst_deg = _pad_slab(dst, E // (NC * NS), DEG_PER_TILE, N_PAD - 1)
    src_m = _pad_slab(src, E // NS, MSG_PER_TILE, 0)
    dst_m = _pad_slab(dst, E // NS, MSG_PER_TILE, N_PAD - 1)
    en_m = _pad_slab(edge_norm, E // NS, MSG_PER_TILE, 0.0)

    degp = _sc_deg_call()(dst_deg).reshape(NC, N_PAD, DW)
    h0, h1 = _tc_linear(feat_p, W.T, b.reshape(1, D), degp)
    agg0, agg1 = _sc_msg_call()(h0, h1, src_m, dst_m, en_m)
    out = _tc_post(agg0, agg1, degp)
    return out[:N]
